# baseline (speedup 1.0000x reference)
import functools

import jax
import jax.numpy as jnp
from jax import lax
from jax.experimental import pallas as pl
from jax.experimental.pallas import tpu as pltpu
from jax.experimental.pallas import tpu_sc as plsc

_V = 1000
_D = 128
_CHUNK = 80


@functools.cache
def _gather_fn(n_tok, off):
    info = plsc.get_sparse_core_info()
    nc, ns = info.num_cores, info.num_subcores
    nw = nc * ns
    per_w = n_tok // nw
    n_steps = per_w // _CHUNK
    assert per_w * nw == n_tok and n_steps * _CHUNK == per_w and n_steps % 2 == 0
    mesh = plsc.VectorSubcoreMesh(core_axis_name="c", subcore_axis_name="s")

    @functools.partial(
        pl.kernel,
        mesh=mesh,
        out_type=jax.ShapeDtypeStruct((n_tok, _D), jnp.float32),
        scratch_types=[
            pltpu.VMEM((per_w,), jnp.int32),
            pltpu.VMEM((_CHUNK, _D), jnp.float32),
            pltpu.VMEM((_CHUNK, _D), jnp.float32),
            pltpu.SemaphoreType.DMA,
            pltpu.SemaphoreType.DMA,
            pltpu.SemaphoreType.DMA,
            pltpu.SemaphoreType.DMA,
        ],
    )
    def gather(table_hbm, ids_hbm, e_hbm, idx_v, buf_a, buf_b, ga, gb, wa, wb):
        wid = lax.axis_index("s") * nc + lax.axis_index("c")
        base = wid * per_w
        pltpu.sync_copy(ids_hbm.at[pl.ds(off + base, per_w)], idx_v)

        def g_issue(i, buf, sem):
            pltpu.async_copy(
                table_hbm.at[idx_v.at[pl.ds(i * _CHUNK, _CHUNK)]], buf, sem
            )

        def g_wait(i, buf, sem):
            pltpu.make_async_copy(
                table_hbm.at[idx_v.at[pl.ds(i * _CHUNK, _CHUNK)]], buf, sem
            ).wait()

        def w_issue(i, buf, sem):
            pltpu.async_copy(buf, e_hbm.at[pl.ds(base + i * _CHUNK, _CHUNK)], sem)

        def w_wait(i, buf, sem):
            pltpu.make_async_copy(
                buf, e_hbm.at[pl.ds(base + i * _CHUNK, _CHUNK)], sem
            ).wait()

        g_issue(0, buf_a, ga)
        g_issue(1, buf_b, gb)

        def body(j, carry):
            ia = 2 * j
            ib = ia + 1
            g_wait(ia, buf_a, ga)
            w_issue(ia, buf_a, wa)
            g_wait(ib, buf_b, gb)
            w_wait(ia, buf_a, wa)

            @pl.when(ia + 2 < n_steps)
            def _():
                g_issue(ia + 2, buf_a, ga)

            w_issue(ib, buf_b, wb)
            w_wait(ib, buf_b, wb)

            @pl.when(ib + 2 < n_steps)
            def _():
                g_issue(ib + 2, buf_b, gb)

            return carry

        lax.fori_loop(0, n_steps // 2, body, 0)

    return gather


def _matmul_body(e_ref, t_ref, o_ref):
    e = e_ref[0].astype(jnp.bfloat16)
    t = t_ref[...].astype(jnp.bfloat16)
    o = lax.dot_general(
        t, e, (((1,), (1,)), ((), ())), preferred_element_type=jnp.float32
    )
    o_ref[0] = o


def _matmul_alias_body(e_ref, t_ref, _p_ref, o_ref):
    _matmul_body(e_ref, t_ref, o_ref)


def _matmul_chunk(e2, table, p, b, l, off, lc):
    if p is None:
        return pl.pallas_call(
            _matmul_body,
            grid=(lc,),
            in_specs=[
                pl.BlockSpec((1, b, _D), lambda i: (i, 0, 0)),
                pl.BlockSpec((_V, _D), lambda i: (0, 0)),
            ],
            out_specs=pl.BlockSpec((1, _V, b), lambda i: (i + off, 0, 0)),
            out_shape=jax.ShapeDtypeStruct((l, _V, b), jnp.float32),
        )(e2, table)
    return pl.pallas_call(
        _matmul_alias_body,
        grid=(lc,),
        in_specs=[
            pl.BlockSpec((1, b, _D), lambda i: (i, 0, 0)),
            pl.BlockSpec((_V, _D), lambda i: (0, 0)),
            pl.BlockSpec(memory_space=pl.ANY),
        ],
        out_specs=pl.BlockSpec((1, _V, b), lambda i: (i + off, 0, 0)),
        out_shape=jax.ShapeDtypeStruct((l, _V, b), jnp.float32),
        input_output_aliases={2: 0},
    )(e2, table, p)


_SPLITS = (5, 10, 15, 20)


def kernel(ids, table):
    b, l = ids.shape
    assert sum(_SPLITS) == l
    idx_all = jnp.transpose(ids).reshape(-1)
    chunks = []
    off = 0
    for lc in _SPLITS:
        e = _gather_fn(lc * b, off * b)(table, idx_all).reshape(lc, b, _D)
        chunks.append((off, lc, e))
        off += lc
    p = None
    for off, lc, e in chunks:
        p = _matmul_chunk(e, table, p, b, l, off, lc)
    return jnp.transpose(p, (2, 0, 1))


# speedup vs baseline: 1.0250x; 1.0250x over previous
import functools

import jax
import jax.numpy as jnp
from jax import lax
from jax.experimental import pallas as pl
from jax.experimental.pallas import tpu as pltpu
from jax.experimental.pallas import tpu_sc as plsc

_V = 1000
_D = 128
_CHUNK = 80


@functools.cache
def _gather_fn(n_tok, off):
    info = plsc.get_sparse_core_info()
    nc, ns = info.num_cores, info.num_subcores
    nw = nc * ns
    per_w = n_tok // nw
    n_steps = per_w // _CHUNK
    assert per_w * nw == n_tok and n_steps * _CHUNK == per_w and n_steps % 2 == 0
    mesh = plsc.VectorSubcoreMesh(core_axis_name="c", subcore_axis_name="s")

    @functools.partial(
        pl.kernel,
        mesh=mesh,
        out_type=jax.ShapeDtypeStruct((n_tok, _D), jnp.float32),
        scratch_types=[
            pltpu.VMEM((per_w,), jnp.int32),
            pltpu.VMEM((_CHUNK, _D), jnp.float32),
            pltpu.VMEM((_CHUNK, _D), jnp.float32),
            pltpu.SemaphoreType.DMA,
            pltpu.SemaphoreType.DMA,
            pltpu.SemaphoreType.DMA,
            pltpu.SemaphoreType.DMA,
        ],
    )
    def gather(table_hbm, ids_hbm, e_hbm, idx_v, buf_a, buf_b, ga, gb, wa, wb):
        wid = lax.axis_index("s") * nc + lax.axis_index("c")
        base = wid * per_w
        pltpu.sync_copy(ids_hbm.at[pl.ds(off + base, per_w)], idx_v)

        def g_issue(i, buf, sem):
            pltpu.async_copy(
                table_hbm.at[idx_v.at[pl.ds(i * _CHUNK, _CHUNK)]], buf, sem
            )

        def g_wait(i, buf, sem):
            pltpu.make_async_copy(
                table_hbm.at[idx_v.at[pl.ds(i * _CHUNK, _CHUNK)]], buf, sem
            ).wait()

        def w_issue(i, buf, sem):
            pltpu.async_copy(buf, e_hbm.at[pl.ds(base + i * _CHUNK, _CHUNK)], sem)

        def w_wait(i, buf, sem):
            pltpu.make_async_copy(
                buf, e_hbm.at[pl.ds(base + i * _CHUNK, _CHUNK)], sem
            ).wait()

        g_issue(0, buf_a, ga)
        g_issue(1, buf_b, gb)

        def body(j, carry):
            ia = 2 * j
            ib = ia + 1
            g_wait(ia, buf_a, ga)
            w_issue(ia, buf_a, wa)
            g_wait(ib, buf_b, gb)
            w_wait(ia, buf_a, wa)

            @pl.when(ia + 2 < n_steps)
            def _():
                g_issue(ia + 2, buf_a, ga)

            w_issue(ib, buf_b, wb)
            w_wait(ib, buf_b, wb)

            @pl.when(ib + 2 < n_steps)
            def _():
                g_issue(ib + 2, buf_b, gb)

            return carry

        lax.fori_loop(0, n_steps // 2, body, 0)

    return gather


def _matmul_body(e_ref, t_ref, o_ref):
    e = e_ref[0].astype(jnp.bfloat16)
    t = t_ref[...].astype(jnp.bfloat16)
    o = lax.dot_general(
        t, e, (((1,), (1,)), ((), ())), preferred_element_type=jnp.float32
    )
    o_ref[0] = o


def _matmul_alias_body(e_ref, t_ref, _p_ref, o_ref):
    _matmul_body(e_ref, t_ref, o_ref)


def _matmul_chunk(e2, table, p, b, l, off, lc):
    if p is None:
        return pl.pallas_call(
            _matmul_body,
            grid=(lc,),
            in_specs=[
                pl.BlockSpec((1, b, _D), lambda i: (i, 0, 0)),
                pl.BlockSpec((_V, _D), lambda i: (0, 0)),
            ],
            out_specs=pl.BlockSpec((1, _V, b), lambda i: (i + off, 0, 0)),
            out_shape=jax.ShapeDtypeStruct((l, _V, b), jnp.float32),
        )(e2, table)
    return pl.pallas_call(
        _matmul_alias_body,
        grid=(lc,),
        in_specs=[
            pl.BlockSpec((1, b, _D), lambda i: (i, 0, 0)),
            pl.BlockSpec((_V, _D), lambda i: (0, 0)),
            pl.BlockSpec(memory_space=pl.ANY),
        ],
        out_specs=pl.BlockSpec((1, _V, b), lambda i: (i + off, 0, 0)),
        out_shape=jax.ShapeDtypeStruct((l, _V, b), jnp.float32),
        input_output_aliases={2: 0},
    )(e2, table, p)


_SPLITS = (10, 15, 25)


def kernel(ids, table):
    b, l = ids.shape
    assert sum(_SPLITS) == l
    idx_all = jnp.transpose(ids).reshape(-1)
    chunks = []
    off = 0
    for lc in _SPLITS:
        e = _gather_fn(lc * b, off * b)(table, idx_all).reshape(lc, b, _D)
        chunks.append((off, lc, e))
        off += lc
    p = None
    for off, lc, e in chunks:
        p = _matmul_chunk(e, table, p, b, l, off, lc)
    return jnp.transpose(p, (2, 0, 1))
